# baseline (speedup 1.0000x reference)
import jax
import jax.numpy as jnp
import numpy as np
from jax.experimental import pallas as pl

EGO = 61
GMS = 275
RES = 0.1
COORD_MIN = -GMS * RES / 2.0
COORD_MAX = GMS * RES / 2.0
LOCAL_SCALE = (COORD_MAX - COORD_MIN) / GMS


def _compute_spatial_locs(depth):
    d = jnp.transpose(depth, (0, 3, 1, 2))
    bs, _, imh, imw = d.shape
    cx, cy = imh / 2.0, imw / 2.0
    fx = imh / 2.0 / np.tan(np.deg2rad(45.0))
    fy = imw / 2.0 / np.tan(np.deg2rad(45.0))
    x = jnp.arange(imw).reshape(1, 1, 1, imw).astype(jnp.float32)
    y = jnp.arange(imh, 0, -1).reshape(1, 1, imh, 1).astype(jnp.float32)
    xx = (x - cx) / fx
    yy = (y - cy) / fy
    Z = d
    X = xx * Z
    Y = yy * Z
    valid = (d != 0) & (Y > -1.5) & (Y < 0.1)
    x_gp = jnp.round(X / LOCAL_SCALE + (EGO - 1) / 2.0).astype(jnp.int32)
    y_gp = jnp.round(-(Z / LOCAL_SCALE) + (EGO - 1) / 2.0).astype(jnp.int32)
    return jnp.concatenate([x_gp, y_gp], axis=1), valid


def _project(conv, spatial_locs, valid_inputs):
    outh, outw = EGO, EGO
    bs, f, HbyK, WbyK = conv.shape
    eps = -1e16
    depth_h = spatial_locs.shape[-1]
    K = depth_h // WbyK
    idx_h = jnp.arange(HbyK) * K
    idx_w = jnp.arange(WbyK) * K
    sl = spatial_locs[:, :, idx_h[:, None], idx_w]
    vi = valid_inputs[:, :, idx_h[:, None], idx_w][:, 0]
    invalid = (~vi) | (sl[:, 1] >= outh) | (sl[:, 1] < 0) | (sl[:, 0] >= outw) | (sl[:, 0] < 0)
    slx = jnp.where(invalid, 0, sl[:, 0])
    sly = jnp.where(invalid, 0, sl[:, 1])
    inv_f = invalid[:, None, :, :].astype(jnp.float32)
    conv_masked = conv * (1.0 - inv_f) + eps * inv_f
    conv_masked = conv_masked.reshape(bs, f, HbyK * WbyK)
    lin = (sly * outw + slx).reshape(bs, HbyK * WbyK)

    def per_b(cm, li):
        res = jax.ops.segment_max(cm.T, li, num_segments=outh * outw)
        return res.T

    proj = jax.vmap(per_b)(conv_masked, lin)
    proj = jnp.where(jnp.isneginf(proj), 0.0, proj)
    eps_mask = (proj == eps).astype(jnp.float32)
    proj = proj * (1.0 - eps_mask) + eps_mask * (proj - eps)
    return proj.reshape(bs, f, outh, outw)


def _affine_grid(A, H, W):
    xs = (jnp.arange(W, dtype=jnp.float32) * 2.0 + 1.0) / W - 1.0
    ys = (jnp.arange(H, dtype=jnp.float32) * 2.0 + 1.0) / H - 1.0
    gx, gy = jnp.meshgrid(xs, ys)
    base = jnp.stack([gx, gy, jnp.ones_like(gx)], axis=-1)
    return jnp.einsum('bij,hwj->bhwi', A, base)


def _grid_sample(im, grid):
    B, C, H, W = im.shape
    ix = ((grid[..., 0] + 1.0) * W - 1.0) / 2.0
    iy = ((grid[..., 1] + 1.0) * H - 1.0) / 2.0
    x0 = jnp.floor(ix)
    y0 = jnp.floor(iy)
    x1 = x0 + 1.0
    y1 = y0 + 1.0
    wa = (x1 - ix) * (y1 - iy)
    wb = (x1 - ix) * (iy - y0)
    wc = (ix - x0) * (y1 - iy)
    wd = (ix - x0) * (iy - y0)

    def gather(xq, yq):
        valid = (xq >= 0) & (xq <= W - 1) & (yq >= 0) & (yq <= H - 1)
        xc = jnp.clip(xq, 0, W - 1).astype(jnp.int32)
        yc = jnp.clip(yq, 0, H - 1).astype(jnp.int32)
        vals = jax.vmap(lambda img, yy, xx: img[:, yy, xx])(im, yc, xc)
        return vals * valid[:, None, :, :].astype(im.dtype)

    out = (wa[:, None] * gather(x0, y0) + wb[:, None] * gather(x0, y1)
           + wc[:, None] * gather(x1, y0) + wd[:, None] * gather(x1, y1))
    return out


def _rotate(x_gp, heading):
    t = heading[:, 0]
    sin_t = jnp.sin(t)
    cos_t = jnp.cos(t)
    z = jnp.zeros_like(t)
    A = jnp.stack([jnp.stack([cos_t, sin_t, z], axis=1),
                   jnp.stack([-sin_t, cos_t, z], axis=1)], axis=1)
    grid = _affine_grid(A, x_gp.shape[2], x_gp.shape[3])
    return _grid_sample(x_gp, grid)


def _identity_body(x_ref, o_ref):
    o_ref[...] = x_ref[...]


def kernel(conv, depth, heading):
    sl, valid = _compute_spatial_locs(depth * 10.0)
    x_gp = _project(conv, sl, valid)
    out = _rotate(x_gp, -heading)
    b, f, h, w = out.shape
    flat = out.reshape(b * f * h, w)
    rows = flat.shape[0]
    blk = 1024
    out2 = pl.pallas_call(
        _identity_body,
        grid=(rows // blk,),
        in_specs=[pl.BlockSpec((blk, w), lambda i: (i, 0))],
        out_specs=pl.BlockSpec((blk, w), lambda i: (i, 0)),
        out_shape=jax.ShapeDtypeStruct(flat.shape, flat.dtype),
    )(flat)
    return out2.reshape(b, f, h, w)


# speedup vs baseline: 1.1981x; 1.1981x over previous
import functools

import jax
import jax.numpy as jnp
import numpy as np
from jax import lax
from jax.experimental import pallas as pl
from jax.experimental.pallas import tpu as pltpu
from jax.experimental.pallas import tpu_sc as plsc

EGO = 61
GMS = 275
RES = 0.1
COORD_MIN = -GMS * RES / 2.0
COORD_MAX = GMS * RES / 2.0
LOCAL_SCALE = (COORD_MAX - COORD_MIN) / GMS

NCELL = EGO * EGO
NPAD = 3840
NPTS = 128 * 128
CHUNK = 1024
NCHUNK = NPTS // CHUNK
NB = 32
NF = 64
FQ = 16
NEG = -1e16


def _prep_body(head_ref, dsub_ref, lin_ref, wts_ref, lidx_ref):
    b = pl.program_id(0)
    Z = dsub_ref[0] * 10.0
    col = lax.broadcasted_iota(jnp.int32, (128, 128), 1).astype(jnp.float32)
    row = lax.broadcasted_iota(jnp.int32, (128, 128), 0).astype(jnp.float32)
    xx = (2.0 * col - 128.0) / 128.0
    yy = (128.0 - 2.0 * row) / 128.0
    X = xx * Z
    Y = yy * Z
    valid = (Z != 0) & (Y > -1.5) & (Y < 0.1)
    x_gp = jnp.round(X / LOCAL_SCALE + (EGO - 1) / 2.0).astype(jnp.int32)
    y_gp = jnp.round(-(Z / LOCAL_SCALE) + (EGO - 1) / 2.0).astype(jnp.int32)
    ok = valid & (y_gp >= 0) & (y_gp < EGO) & (x_gp >= 0) & (x_gp < EGO)
    lin_ref[0] = jnp.where(ok, y_gp * EGO + x_gp, -1)

    t = -head_ref[b, 0]
    tb = jnp.full((30, 128), t, jnp.float32)
    sin_t = jnp.sin(tb)
    cos_t = jnp.cos(tb)
    cell = (lax.broadcasted_iota(jnp.int32, (30, 128), 0) * 128
            + lax.broadcasted_iota(jnp.int32, (30, 128), 1))
    cellf = cell.astype(jnp.float32)
    ii = jnp.floor((cellf + 0.5) / float(EGO))
    jj = cellf - ii * float(EGO)
    inb = cell < NCELL
    gx = (jj * 2.0 + 1.0) / EGO - 1.0
    gy = (ii * 2.0 + 1.0) / EGO - 1.0
    def _bf(x):
        return x.astype(jnp.bfloat16).astype(jnp.float32)
    gxb, gyb = _bf(gx), _bf(gy)
    sinb, cosb = _bf(sin_t), _bf(cos_t)
    gridx = cosb * gxb + sinb * gyb
    gridy = (-sinb) * gxb + cosb * gyb
    ix = ((gridx + 1.0) * EGO - 1.0) / 2.0
    iy = ((gridy + 1.0) * EGO - 1.0) / 2.0
    x0 = jnp.floor(ix)
    y0 = jnp.floor(iy)
    x1 = x0 + 1.0
    y1 = y0 + 1.0
    corners = (
        (x0, y0, (x1 - ix) * (y1 - iy)),
        (x0, y1, (x1 - ix) * (iy - y0)),
        (x1, y0, (ix - x0) * (y1 - iy)),
        (x1, y1, (ix - x0) * (iy - y0)),
    )
    for k, (xq, yq, w) in enumerate(corners):
        v = (xq >= 0) & (xq <= EGO - 1) & (yq >= 0) & (yq <= EGO - 1) & inb
        xc = jnp.clip(xq, 0, EGO - 1).astype(jnp.int32)
        yc = jnp.clip(yq, 0, EGO - 1).astype(jnp.int32)
        wts_ref[0, k] = w * v.astype(jnp.float32)
        lidx_ref[0, k] = yc * EGO + xc


def _tc_prep(depth_sub, heading):
    return pl.pallas_call(
        _prep_body,
        grid=(NB,),
        in_specs=[
            pl.BlockSpec(memory_space=pltpu.SMEM),
            pl.BlockSpec((1, 128, 128), lambda b: (b, 0, 0)),
        ],
        out_specs=[
            pl.BlockSpec((1, 128, 128), lambda b: (b, 0, 0)),
            pl.BlockSpec((1, 4, 30, 128), lambda b: (b, 0, 0, 0)),
            pl.BlockSpec((1, 4, 30, 128), lambda b: (b, 0, 0, 0)),
        ],
        out_shape=[
            jax.ShapeDtypeStruct((NB, 128, 128), jnp.int32),
            jax.ShapeDtypeStruct((NB, 4, 30, 128), jnp.float32),
            jax.ShapeDtypeStruct((NB, 4, 30, 128), jnp.int32),
        ],
    )(heading, depth_sub)


def _sc_scatter_body(conv_hbm, lin_hbm, neg_hbm, proj_hbm,
                     lin_all, lin_cmp, pid_cmp, projq, cbuf, bnd, sem):
    cid = lax.axis_index("c")
    sid = lax.axis_index("s")
    b = cid * 16 + sid
    iota = lax.iota(jnp.int32, 16)

    pltpu.sync_copy(lin_hbm.at[b], lin_all)
    pltpu.sync_copy(neg_hbm, projq)

    bnd[0] = 0

    def chunk_step(ci, cnt):
        def vreg_step(vi, cnt):
            base = ci * CHUNK + vi * 16
            lvec = plsc.load_gather(lin_all, [base + iota])
            mask = lvec >= 0
            m32 = mask.astype(jnp.int32)
            pos = cnt + plsc.cumsum(m32) - 1
            plsc.store_scatter(lin_cmp, [pos], lvec, mask=mask)
            plsc.store_scatter(pid_cmp, [pos], vi * 16 + iota, mask=mask)
            return cnt + jnp.sum(m32)

        cnt = lax.fori_loop(0, CHUNK // 16, vreg_step, cnt)
        bnd[ci + 1] = cnt
        return cnt

    lax.fori_loop(0, NCHUNK, chunk_step, 0)

    for fq in range(4):
        def chunk_scatter(ci, _):
            pltpu.sync_copy(
                conv_hbm.at[b, pl.ds(fq * FQ, FQ), pl.ds(ci * CHUNK, CHUNK)],
                cbuf)
            lo = bnd[ci]
            hi = bnd[ci + 1]

            def pt(i, _):
                cellv = plsc.load_gather(lin_cmp, [jnp.full((16,), i, jnp.int32)])
                offv = plsc.load_gather(pid_cmp, [jnp.full((16,), i, jnp.int32)])
                vals = plsc.load_gather(cbuf, [iota, offv])
                cur = plsc.load_gather(projq, [cellv, iota])
                plsc.store_scatter(projq, [cellv, iota], jnp.maximum(cur, vals))
                return 0

            lax.fori_loop(lo, hi, pt, 0)
            return 0

        lax.fori_loop(0, NCHUNK, chunk_scatter, 0)

        def fix(i, _):
            rv = jnp.full((16,), i, jnp.int32)
            v = plsc.load_gather(projq, [rv, iota])
            plsc.store_scatter(projq, [rv, iota],
                               jnp.where(v == NEG, 0.0, v))
            return 0

        lax.fori_loop(0, NCELL, fix, 0)
        pltpu.sync_copy(projq, proj_hbm.at[b, fq])
        if fq != 3:
            pltpu.sync_copy(neg_hbm, projq)


def _sc_scatter(conv_flat, lin, neg_const):
    mesh = plsc.VectorSubcoreMesh(core_axis_name="c", subcore_axis_name="s")
    f = functools.partial(
        pl.kernel,
        mesh=mesh,
        compiler_params=pltpu.CompilerParams(needs_layout_passes=False, use_tc_tiling_on_sc=False),
        out_type=jax.ShapeDtypeStruct((NB, 4, NCELL, FQ), jnp.float32),
        scratch_types=[
            pltpu.VMEM((NPTS,), jnp.int32),
            pltpu.VMEM((NPTS + 16,), jnp.int32),
            pltpu.VMEM((NPTS + 16,), jnp.int32),
            pltpu.VMEM((NCELL, FQ), jnp.float32),
            pltpu.VMEM((FQ, CHUNK), jnp.float32),
            pltpu.SMEM((NCHUNK + 1,), jnp.int32),
            pltpu.SemaphoreType.DMA,
        ],
    )(_sc_scatter_body)
    return f(conv_flat, lin, neg_const)


def _sc_sample_body(proj_hbm, wts_hbm, lidx_hbm, out_hbm,
                    idxb, wb, rows, obuf, sem):
    cid = lax.axis_index("c")
    sid = lax.axis_index("s")
    b = cid * 16 + sid
    iota = lax.iota(jnp.int32, 16)

    def group(g, _):
        for k in range(4):
            pltpu.sync_copy(lidx_hbm.at[b, k, pl.ds(g * 128, 128)], idxb.at[k])
            pltpu.sync_copy(wts_hbm.at[b, k, pl.ds(g * 128, 128)], wb.at[k])
        handles = []
        for k in range(4):
            for fq in range(4):
                handles.append(pltpu.async_copy(
                    proj_hbm.at[b, fq].at[idxb.at[k]], rows.at[k, fq], sem))
        for h in handles:
            h.wait()

        def cellstep(cl, _):
            clv = jnp.full((16,), cl, jnp.int32)
            ws = [plsc.load_gather(wb, [jnp.full((16,), k, jnp.int32), clv])
                  for k in range(4)]
            for fq in range(4):
                acc = jnp.zeros((16,), jnp.float32)
                for k in range(4):
                    r = plsc.load_gather(
                        rows, [jnp.full((16,), k, jnp.int32),
                               jnp.full((16,), fq, jnp.int32), clv, iota])
                    acc = acc + ws[k] * r
                plsc.store_scatter(obuf, [fq * 16 + iota, clv], acc)
            return 0

        lax.fori_loop(0, 128, cellstep, 0)
        pltpu.sync_copy(obuf, out_hbm.at[b, :, pl.ds(g * 128, 128)])
        return 0

    lax.fori_loop(0, NPAD // 128, group, 0)


def _sc_sample(proj, wts, lidx):
    mesh = plsc.VectorSubcoreMesh(core_axis_name="c", subcore_axis_name="s")
    f = functools.partial(
        pl.kernel,
        mesh=mesh,
        compiler_params=pltpu.CompilerParams(needs_layout_passes=False, use_tc_tiling_on_sc=False),
        out_type=jax.ShapeDtypeStruct((NB, NF, NPAD), jnp.float32),
        scratch_types=[
            pltpu.VMEM((4, 128), jnp.int32),
            pltpu.VMEM((4, 128), jnp.float32),
            pltpu.VMEM((4, 4, 128, FQ), jnp.float32),
            pltpu.VMEM((NF, 128), jnp.float32),
            pltpu.SemaphoreType.DMA,
        ],
    )(_sc_sample_body)
    return f(proj, wts, lidx)


def kernel(conv, depth, heading):
    depth_sub = depth[:, ::2, ::2, 0]
    lin, wts, lidx = _tc_prep(depth_sub, heading)
    lin = lin.reshape(NB, NPTS)
    wts = wts.reshape(NB, 4, NPAD)
    lidx = lidx.reshape(NB, 4, NPAD)
    conv_flat = conv.reshape(NB, NF, NPTS)
    neg_const = jnp.full((NCELL, FQ), NEG, jnp.float32)
    proj = _sc_scatter(conv_flat, lin, neg_const)
    out = _sc_sample(proj, wts, lidx)
    return out[:, :, :NCELL].reshape(NB, NF, EGO, EGO)


# speedup vs baseline: 3.0468x; 2.5431x over previous
import functools

import jax
import jax.numpy as jnp
import numpy as np
from jax import lax
from jax.experimental import pallas as pl
from jax.experimental.pallas import tpu as pltpu
from jax.experimental.pallas import tpu_sc as plsc

EGO = 61
GMS = 275
RES = 0.1
COORD_MIN = -GMS * RES / 2.0
COORD_MAX = GMS * RES / 2.0
LOCAL_SCALE = (COORD_MAX - COORD_MIN) / GMS

NCELL = EGO * EGO
NPAD = 3840
NPTS = 256 * 256
STAGE = 4096
CH = 1024
NCH = NPTS // CH
CCH = 256
NB = 32
NF = 64
FQ = 16
NEG = -1e16


def _prep_body(head_ref, dfull_ref, lin_ref, wts_ref, lidx_ref):
    b = pl.program_id(0)
    Z = dfull_ref[0] * 10.0
    coli = lax.broadcasted_iota(jnp.int32, (256, 256), 1)
    rowi = lax.broadcasted_iota(jnp.int32, (256, 256), 0)
    col = coli.astype(jnp.float32)
    row = rowi.astype(jnp.float32)
    xx = (col - 128.0) / 128.0
    yy = (128.0 - row) / 128.0
    X = xx * Z
    Y = yy * Z
    even = ((rowi & 1) == 0) & ((coli & 1) == 0)
    valid = (Z != 0) & (Y > -1.5) & (Y < 0.1) & even
    x_gp = jnp.round(X / LOCAL_SCALE + (EGO - 1) / 2.0).astype(jnp.int32)
    y_gp = jnp.round(-(Z / LOCAL_SCALE) + (EGO - 1) / 2.0).astype(jnp.int32)
    ok = valid & (y_gp >= 0) & (y_gp < EGO) & (x_gp >= 0) & (x_gp < EGO)
    lin_ref[0] = jnp.where(ok, y_gp * EGO + x_gp, -1)

    t = -head_ref[b, 0]
    tb = jnp.full((30, 128), t, jnp.float32)
    sin_t = jnp.sin(tb)
    cos_t = jnp.cos(tb)
    cell = (lax.broadcasted_iota(jnp.int32, (30, 128), 0) * 128
            + lax.broadcasted_iota(jnp.int32, (30, 128), 1))
    cellf = cell.astype(jnp.float32)
    ii = jnp.floor((cellf + 0.5) / float(EGO))
    jj = cellf - ii * float(EGO)
    inb = cell < NCELL
    gx = (jj * 2.0 + 1.0) / EGO - 1.0
    gy = (ii * 2.0 + 1.0) / EGO - 1.0
    def _bf(x):
        return x.astype(jnp.bfloat16).astype(jnp.float32)
    gxb, gyb = _bf(gx), _bf(gy)
    sinb, cosb = _bf(sin_t), _bf(cos_t)
    gridx = cosb * gxb + sinb * gyb
    gridy = (-sinb) * gxb + cosb * gyb
    ix = ((gridx + 1.0) * EGO - 1.0) / 2.0
    iy = ((gridy + 1.0) * EGO - 1.0) / 2.0
    x0 = jnp.floor(ix)
    y0 = jnp.floor(iy)
    x1 = x0 + 1.0
    y1 = y0 + 1.0
    corners = (
        (x0, y0, (x1 - ix) * (y1 - iy)),
        (x0, y1, (x1 - ix) * (iy - y0)),
        (x1, y0, (ix - x0) * (y1 - iy)),
        (x1, y1, (ix - x0) * (iy - y0)),
    )
    for k, (xq, yq, w) in enumerate(corners):
        v = (xq >= 0) & (xq <= EGO - 1) & (yq >= 0) & (yq <= EGO - 1) & inb
        xc = jnp.clip(xq, 0, EGO - 1).astype(jnp.int32)
        yc = jnp.clip(yq, 0, EGO - 1).astype(jnp.int32)
        wts_ref[0, k] = w * v.astype(jnp.float32)
        lidx_ref[0, k] = yc * EGO + xc


def _tc_prep(depth_full, heading):
    return pl.pallas_call(
        _prep_body,
        grid=(NB,),
        in_specs=[
            pl.BlockSpec(memory_space=pltpu.SMEM),
            pl.BlockSpec((1, 256, 256), lambda b: (b, 0, 0)),
        ],
        out_specs=[
            pl.BlockSpec((1, 256, 256), lambda b: (b, 0, 0)),
            pl.BlockSpec((1, 4, 30, 128), lambda b: (b, 0, 0, 0)),
            pl.BlockSpec((1, 4, 30, 128), lambda b: (b, 0, 0, 0)),
        ],
        out_shape=[
            jax.ShapeDtypeStruct((NB, 256, 256), jnp.int32),
            jax.ShapeDtypeStruct((NB, 4, 30, 128), jnp.float32),
            jax.ShapeDtypeStruct((NB, 4, 30, 128), jnp.int32),
        ],
    )(heading, depth_full)


def _sc_scatter_body(conv_hbm, lin_hbm, neg_hbm, proj_hbm,
                     stage, lin_cmp, pid_cmp, projq, cbuf, bnd, sem):
    cid = lax.axis_index("c")
    sid = lax.axis_index("s")
    b = cid * 16 + sid
    iota = lax.iota(jnp.int32, 16)

    pltpu.sync_copy(neg_hbm, projq)

    bnd[0] = 0

    def stage_step(sc, cnt):
        pltpu.sync_copy(lin_hbm.at[b, pl.ds(sc * STAGE, STAGE)], stage)

        def chunk_step(q, cnt):
            def vreg_step(vi, cnt):
                off = q * CH + vi * 16
                lvec = plsc.load_gather(stage, [off + iota])
                mask = lvec >= 0
                m32 = mask.astype(jnp.int32)
                pos = cnt + plsc.cumsum(m32) - 1
                o256 = (vi * 16 + iota)
                rloc = lax.shift_right_logical(o256, 8)
                c256 = o256 & 255
                coff = (lax.shift_right_logical(rloc, 1) * 128
                        + lax.shift_right_logical(c256, 1))
                plsc.store_scatter(lin_cmp, [pos], lvec, mask=mask)
                plsc.store_scatter(pid_cmp, [pos], coff, mask=mask)
                return cnt + jnp.sum(m32)

            cnt = lax.fori_loop(0, CH // 16, vreg_step, cnt)
            bnd[sc * (STAGE // CH) + q + 1] = cnt
            return cnt

        return lax.fori_loop(0, STAGE // CH, chunk_step, cnt)

    lax.fori_loop(0, NPTS // STAGE, stage_step, 0)

    for fq in range(4):
        def chunk_scatter(ci, _):
            pltpu.sync_copy(
                conv_hbm.at[b, pl.ds(fq * FQ, FQ), pl.ds(ci * CCH, CCH)],
                cbuf)
            lo = bnd[ci]
            hi = bnd[ci + 1]

            def pt(i, _):
                cellv = plsc.load_gather(lin_cmp, [jnp.full((16,), i, jnp.int32)])
                offv = plsc.load_gather(pid_cmp, [jnp.full((16,), i, jnp.int32)])
                vals = plsc.load_gather(cbuf, [iota, offv])
                cur = plsc.load_gather(projq, [cellv, iota])
                plsc.store_scatter(projq, [cellv, iota], jnp.maximum(cur, vals))
                return 0

            lax.fori_loop(lo, hi, pt, 0)
            return 0

        lax.fori_loop(0, NCH, chunk_scatter, 0)

        def fix(i, _):
            rv = jnp.full((16,), i, jnp.int32)
            v = plsc.load_gather(projq, [rv, iota])
            plsc.store_scatter(projq, [rv, iota],
                               jnp.where(v == NEG, 0.0, v))
            return 0

        lax.fori_loop(0, NCELL, fix, 0)
        pltpu.sync_copy(projq, proj_hbm.at[b, fq])
        if fq != 3:
            pltpu.sync_copy(neg_hbm, projq)


def _sc_scatter(conv_flat, lin, neg_const):
    mesh = plsc.VectorSubcoreMesh(core_axis_name="c", subcore_axis_name="s")
    f = functools.partial(
        pl.kernel,
        mesh=mesh,
        compiler_params=pltpu.CompilerParams(needs_layout_passes=False, use_tc_tiling_on_sc=False),
        out_type=jax.ShapeDtypeStruct((NB, 4, NCELL, FQ), jnp.float32),
        scratch_types=[
            pltpu.VMEM((STAGE,), jnp.int32),
            pltpu.VMEM((NPTS // 4 + 16,), jnp.int32),
            pltpu.VMEM((NPTS // 4 + 16,), jnp.int32),
            pltpu.VMEM((NCELL, FQ), jnp.float32),
            pltpu.VMEM((FQ, CCH), jnp.float32),
            pltpu.SMEM((NCH + 1,), jnp.int32),
            pltpu.SemaphoreType.DMA,
        ],
    )(_sc_scatter_body)
    return f(conv_flat, lin, neg_const)


def _sc_sample_body(proj_hbm, wts_hbm, lidx_hbm, out_hbm,
                    idxb, wb, rows, obuf, sem):
    cid = lax.axis_index("c")
    sid = lax.axis_index("s")
    b = cid * 16 + sid
    iota = lax.iota(jnp.int32, 16)

    def group(g, _):
        for k in range(4):
            pltpu.sync_copy(lidx_hbm.at[b, k, pl.ds(g * 128, 128)], idxb.at[k])
            pltpu.sync_copy(wts_hbm.at[b, k, pl.ds(g * 128, 128)], wb.at[k])
        handles = []
        for k in range(4):
            for fq in range(4):
                handles.append(pltpu.async_copy(
                    proj_hbm.at[b, fq].at[idxb.at[k]], rows.at[k, fq], sem))
        for h in handles:
            h.wait()

        def cellstep(cl, _):
            clv = jnp.full((16,), cl, jnp.int32)
            ws = [plsc.load_gather(wb, [jnp.full((16,), k, jnp.int32), clv])
                  for k in range(4)]
            for fq in range(4):
                acc = jnp.zeros((16,), jnp.float32)
                for k in range(4):
                    r = plsc.load_gather(
                        rows, [jnp.full((16,), k, jnp.int32),
                               jnp.full((16,), fq, jnp.int32), clv, iota])
                    acc = acc + ws[k] * r
                plsc.store_scatter(obuf, [fq * 16 + iota, clv], acc)
            return 0

        lax.fori_loop(0, 128, cellstep, 0)
        pltpu.sync_copy(obuf, out_hbm.at[b, :, pl.ds(g * 128, 128)])
        return 0

    lax.fori_loop(0, NPAD // 128, group, 0)


def _sc_sample(proj, wts, lidx):
    mesh = plsc.VectorSubcoreMesh(core_axis_name="c", subcore_axis_name="s")
    f = functools.partial(
        pl.kernel,
        mesh=mesh,
        compiler_params=pltpu.CompilerParams(needs_layout_passes=False, use_tc_tiling_on_sc=False),
        out_type=jax.ShapeDtypeStruct((NB, NF, NPAD), jnp.float32),
        scratch_types=[
            pltpu.VMEM((4, 128), jnp.int32),
            pltpu.VMEM((4, 128), jnp.float32),
            pltpu.VMEM((4, 4, 128, FQ), jnp.float32),
            pltpu.VMEM((NF, 128), jnp.float32),
            pltpu.SemaphoreType.DMA,
        ],
    )(_sc_sample_body)
    return f(proj, wts, lidx)


def kernel(conv, depth, heading):
    depth_full = depth.reshape(NB, 256, 256)
    lin, wts, lidx = _tc_prep(depth_full, heading)
    lin = lin.reshape(NB, NPTS)
    wts = wts.reshape(NB, 4, NPAD)
    lidx = lidx.reshape(NB, 4, NPAD)
    conv_flat = conv.reshape(NB, NF, NPTS // 4)
    neg_const = jnp.full((NCELL, FQ), NEG, jnp.float32)
    proj = _sc_scatter(conv_flat, lin, neg_const)
    out = _sc_sample(proj, wts, lidx)
    return out[:, :, :NCELL].reshape(NB, NF, EGO, EGO)
